# Initial kernel scaffold; baseline (speedup 1.0000x reference)
import jax
import jax.numpy as jnp
from jax.experimental import pallas as pl


def kernel(x, edge_attr, W1, a_src1, a_dst1, b1, W2, a_src2, a_dst2, b2, Wih, Whh, bih, bhh, edge_index):
    raise NotImplementedError("write your pallas kernel here")


# speedup vs baseline: 3.1433x; 3.1433x over previous
import functools

import jax
import jax.numpy as jnp
from jax import lax
from jax.experimental import pallas as pl
from jax.experimental.pallas import tpu as pltpu

N = 10000
E = 320000
D = 128
H = 8
T = 4
C = 16
NB = 400
NBLK = N // NB

_IP = False


def _k1_body(x_ref, w1_ref, as_ref, ad_ref, h1_ref, a_ref):
    xb = x_ref[...]
    for h in range(H):
        w = w1_ref[:, h * D:(h + 1) * D]
        hh = jnp.dot(xb, w, preferred_element_type=jnp.float32)
        h1_ref[h] = hh
        a_ref[:, h:h + 1] = jnp.sum(hh * as_ref[h][None, :], axis=1)[:, None]
        a_ref[:, 8 + h:9 + h] = jnp.sum(hh * ad_ref[h][None, :], axis=1)[:, None]


def _k1(x_t, W1, a_src1, a_dst1):
    return pl.pallas_call(
        _k1_body,
        grid=(NBLK,),
        in_specs=[
            pl.BlockSpec((NB, D), lambda i: (i, 0)),
            pl.BlockSpec((D, H * D), lambda i: (0, 0)),
            pl.BlockSpec((H, D), lambda i: (0, 0)),
            pl.BlockSpec((H, D), lambda i: (0, 0)),
        ],
        out_specs=[
            pl.BlockSpec((H, NB, D), lambda i: (0, i, 0)),
            pl.BlockSpec((NB, 16), lambda i: (i, 0)),
        ],
        out_shape=[
            jax.ShapeDtypeStruct((H, N, D), jnp.float32),
            jax.ShapeDtypeStruct((N, 16), jnp.float32),
        ],
        interpret=_IP,
    )(x_t, W1, a_src1, a_dst1)


def _k2_body(acc_ref, den_ref, b1_ref, w2_ref, as2_ref, ad2_ref,
             h2_ref, a2_ref):
    den = den_ref[0] + den_ref[1]
    acc = jnp.zeros((NB, D), dtype=jnp.float32)
    for h in range(H):
        z = acc_ref[h] / den[:, h:h + 1] + b1_ref[pl.ds(h * D, D)][None, :]
        z = jnp.maximum(z, 0.0)
        acc = acc + jnp.dot(z, w2_ref[pl.ds(h * D, D), :],
                            preferred_element_type=jnp.float32)
    h2_ref[...] = acc
    a2_ref[:, 0:1] = jnp.sum(acc * as2_ref[0][None, :], axis=1)[:, None]
    a2_ref[:, 8:9] = jnp.sum(acc * ad2_ref[0][None, :], axis=1)[:, None]


def _k2(acc1, den1, b1, W2, a_src2, a_dst2):
    return pl.pallas_call(
        _k2_body,
        grid=(NBLK,),
        in_specs=[
            pl.BlockSpec((H, NB, D), lambda i: (0, i, 0)),
            pl.BlockSpec((2, NB, 16), lambda i: (0, i, 0)),
            pl.BlockSpec((H * D,), lambda i: (0,)),
            pl.BlockSpec((H * D, D), lambda i: (0, 0)),
            pl.BlockSpec((1, D), lambda i: (0, 0)),
            pl.BlockSpec((1, D), lambda i: (0, 0)),
        ],
        out_specs=[
            pl.BlockSpec((NB, D), lambda i: (i, 0)),
            pl.BlockSpec((NB, 16), lambda i: (i, 0)),
        ],
        out_shape=[
            jax.ShapeDtypeStruct((N, D), jnp.float32),
            jax.ShapeDtypeStruct((N, 16), jnp.float32),
        ],
        interpret=_IP,
    )(acc1, den1, b1, W2, a_src2, a_dst2)


def _k3_body(*refs):
    acc_refs = refs[0:T]
    den_refs = refs[T:2 * T]
    b2_ref, wih_ref, whh_ref, bi_ref, bh_ref = refs[2 * T:2 * T + 5]
    hseq_ref, p_ref = refs[2 * T + 5:]
    hst = jnp.zeros((NB, C), dtype=jnp.float32)
    cst = jnp.zeros((NB, C), dtype=jnp.float32)
    bias = (bi_ref[...] + bh_ref[...])[None, :]
    for t in range(T):
        den = den_refs[t][0] + den_refs[t][1]
        hs = acc_refs[t][...] / den[:, 0:1] + b2_ref[...][None, :]
        gates = (jnp.dot(hs, wih_ref[...], preferred_element_type=jnp.float32)
                 + jnp.dot(hst, whh_ref[...], preferred_element_type=jnp.float32)
                 + bias)
        i = jax.nn.sigmoid(gates[:, 0:C])
        f = jax.nn.sigmoid(gates[:, C:2 * C])
        g = jnp.tanh(gates[:, 2 * C:3 * C])
        o = jax.nn.sigmoid(gates[:, 3 * C:4 * C])
        cst = f * cst + i * g
        hst = o * jnp.tanh(cst)
        hseq_ref[t] = hst
        p_ref[0, t] = jnp.sum(jnp.exp(hst), axis=0)


def _k3(acc2s, den2s, b2, WihT, WhhT, bih, bhh):
    in_specs = (
        [pl.BlockSpec((NB, D), lambda i: (i, 0)) for _ in range(T)]
        + [pl.BlockSpec((2, NB, 16), lambda i: (0, i, 0)) for _ in range(T)]
        + [
            pl.BlockSpec((D,), lambda i: (0,)),
            pl.BlockSpec((D, 4 * C), lambda i: (0, 0)),
            pl.BlockSpec((C, 4 * C), lambda i: (0, 0)),
            pl.BlockSpec((4 * C,), lambda i: (0,)),
            pl.BlockSpec((4 * C,), lambda i: (0,)),
        ])
    return pl.pallas_call(
        _k3_body,
        grid=(NBLK,),
        in_specs=in_specs,
        out_specs=[
            pl.BlockSpec((T, NB, C), lambda i: (0, i, 0)),
            pl.BlockSpec((1, T, C), lambda i: (i, 0, 0)),
        ],
        out_shape=[
            jax.ShapeDtypeStruct((T, N, C), jnp.float32),
            jax.ShapeDtypeStruct((NBLK, T, C), jnp.float32),
        ],
        interpret=_IP,
    )(*acc2s, *den2s, b2, WihT, WhhT, bih, bhh)


def _k4_body(h_ref, p_ref, o_ref):
    tot = jnp.sum(p_ref[...], axis=0)[None, :, :]
    o_ref[...] = jnp.exp(h_ref[...]) / tot.transpose(1, 0, 2)


def _k4(hseq, p):
    return pl.pallas_call(
        _k4_body,
        grid=(NBLK,),
        in_specs=[
            pl.BlockSpec((T, NB, C), lambda i: (0, i, 0)),
            pl.BlockSpec((NBLK, T, C), lambda i: (0, 0, 0)),
        ],
        out_specs=pl.BlockSpec((T, NB, C), lambda i: (0, i, 0)),
        out_shape=jax.ShapeDtypeStruct((T, N, C), jnp.float32),
        interpret=_IP,
    )(hseq, p)


def _edge_layer(h_flat, a_nodes, src, dst, nh):
    asrc = a_nodes[:, 0:nh]
    adst = a_nodes[:, 8:8 + nh]
    e = jax.nn.leaky_relu(asrc[src] + adst[dst], 0.2)
    ex = jnp.exp(e)
    den = jax.ops.segment_sum(ex, dst, num_segments=N)
    den16 = jnp.zeros((N, 16), jnp.float32).at[:, 0:nh].set(den)
    accs = []
    for h in range(nh):
        msg = h_flat[h * N:(h + 1) * N][src] * ex[:, h][:, None]
        accs.append(jax.ops.segment_sum(msg, dst, num_segments=N))
    acc = jnp.stack(accs, axis=0)
    return acc, jnp.stack([den16, jnp.zeros_like(den16)], axis=0)


def kernel(x, edge_attr, W1, a_src1, a_dst1, b1, W2, a_src2, a_dst2, b2,
           Wih, Whh, bih, bhh, edge_index):
    loop = jnp.arange(N, dtype=edge_index.dtype)
    src = jnp.concatenate([edge_index[0], loop])
    dst = jnp.concatenate([edge_index[1], loop])

    acc2s, den2s = [], []
    for t in range(T):
        h1, a1 = _k1(x[t], W1, a_src1, a_dst1)
        acc1, den1 = _edge_layer(h1.reshape(H * N, D), a1, src, dst, H)
        h2, a2 = _k2(acc1, den1, b1, W2, a_src2, a_dst2)
        acc2, den2 = _edge_layer(h2, a2, src, dst, 1)
        acc2s.append(acc2[0])
        den2s.append(den2)

    hseq, p = _k3(acc2s, den2s, b2, Wih.T, Whh.T, bih, bhh)
    return _k4(hseq, p)


# speedup vs baseline: 10.1364x; 3.2247x over previous
import functools

import jax
import jax.numpy as jnp
from jax import lax
from jax.experimental import pallas as pl
from jax.experimental.pallas import tpu as pltpu
from jax.experimental.pallas import tpu_sc as plsc

N = 10000
E = 320000
D = 128
H = 8
T = 4
C = 16
NB = 400
NBLK = N // NB

NC, NS, L = 2, 16, 16
KCH = 128
ETP = 331776
NCH = ETP // KCH
CPT_A = NCH // (NC * NS)
CPT_B = NCH // NS
NP1 = 10240
NH2 = 5120

_sc_mesh = plsc.VectorSubcoreMesh(core_axis_name="c", subcore_axis_name="s")


def _k1_body(x_ref, w1_ref, as_ref, ad_ref, h1_ref, s_ref, d_ref):
    xb = x_ref[...]
    s_ref[...] = jnp.zeros((NB, 16), jnp.float32)
    d_ref[...] = jnp.zeros((NB, 16), jnp.float32)
    for h in range(H):
        w = w1_ref[:, h * D:(h + 1) * D]
        hh = jnp.dot(xb, w, preferred_element_type=jnp.float32)
        h1_ref[h] = hh
        s_ref[:, h:h + 1] = jnp.sum(hh * as_ref[h][None, :], axis=1)[:, None]
        d_ref[:, h:h + 1] = jnp.sum(hh * ad_ref[h][None, :], axis=1)[:, None]


def _k1(x_t, W1, a_src1, a_dst1):
    return pl.pallas_call(
        _k1_body,
        grid=(NBLK,),
        in_specs=[
            pl.BlockSpec((NB, D), lambda i: (i, 0)),
            pl.BlockSpec((D, H * D), lambda i: (0, 0)),
            pl.BlockSpec((H, D), lambda i: (0, 0)),
            pl.BlockSpec((H, D), lambda i: (0, 0)),
        ],
        out_specs=[
            pl.BlockSpec((H, NB, D), lambda i: (0, i, 0)),
            pl.BlockSpec((NB, 16), lambda i: (i, 0)),
            pl.BlockSpec((NB, 16), lambda i: (i, 0)),
        ],
        out_shape=[
            jax.ShapeDtypeStruct((H, N, D), jnp.float32),
            jax.ShapeDtypeStruct((N, 16), jnp.float32),
            jax.ShapeDtypeStruct((N, 16), jnp.float32),
        ],
    )(x_t, W1, a_src1, a_dst1)


def _k2_body(acc_ref, den_ref, b1_ref, w2_ref, as2_ref, ad2_ref,
             h2_ref, s_ref, d_ref):
    den = den_ref[0] + den_ref[1]
    acc = jnp.zeros((NB, D), dtype=jnp.float32)
    for h in range(H):
        z = ((acc_ref[0, h] + acc_ref[1, h]) / den[:, h:h + 1]
             + b1_ref[pl.ds(h * D, D)][None, :])
        z = jnp.maximum(z, 0.0)
        acc = acc + jnp.dot(z, w2_ref[pl.ds(h * D, D), :],
                            preferred_element_type=jnp.float32)
    h2_ref[...] = acc
    s_ref[...] = jnp.zeros((NB, 16), jnp.float32)
    d_ref[...] = jnp.zeros((NB, 16), jnp.float32)
    s_ref[:, 0:1] = jnp.sum(acc * as2_ref[0][None, :], axis=1)[:, None]
    d_ref[:, 0:1] = jnp.sum(acc * ad2_ref[0][None, :], axis=1)[:, None]


def _k2(acc1, den1, b1, W2, a_src2, a_dst2):
    return pl.pallas_call(
        _k2_body,
        grid=(NBLK,),
        in_specs=[
            pl.BlockSpec((2, H, NB, D), lambda i: (0, 0, i, 0)),
            pl.BlockSpec((2, NB, 16), lambda i: (0, i, 0)),
            pl.BlockSpec((H * D,), lambda i: (0,)),
            pl.BlockSpec((H * D, D), lambda i: (0, 0)),
            pl.BlockSpec((1, D), lambda i: (0, 0)),
            pl.BlockSpec((1, D), lambda i: (0, 0)),
        ],
        out_specs=[
            pl.BlockSpec((NB, D), lambda i: (i, 0)),
            pl.BlockSpec((NB, 16), lambda i: (i, 0)),
            pl.BlockSpec((NB, 16), lambda i: (i, 0)),
        ],
        out_shape=[
            jax.ShapeDtypeStruct((N, D), jnp.float32),
            jax.ShapeDtypeStruct((N, 16), jnp.float32),
            jax.ShapeDtypeStruct((N, 16), jnp.float32),
        ],
    )(acc1, den1, b1, W2, a_src2, a_dst2)


def _k3_body(*refs):
    acc_refs = refs[0:T]
    den_refs = refs[T:2 * T]
    b2_ref, wih_ref, whh_ref, bi_ref, bh_ref = refs[2 * T:2 * T + 5]
    hseq_ref, p_ref = refs[2 * T + 5:]
    hst = jnp.zeros((NB, C), dtype=jnp.float32)
    cst = jnp.zeros((NB, C), dtype=jnp.float32)
    bias = (bi_ref[...] + bh_ref[...])[None, :]
    for t in range(T):
        den = den_refs[t][0] + den_refs[t][1]
        hs = acc_refs[t][...] / den[:, 0:1] + b2_ref[...][None, :]
        gates = (jnp.dot(hs, wih_ref[...], preferred_element_type=jnp.float32)
                 + jnp.dot(hst, whh_ref[...], preferred_element_type=jnp.float32)
                 + bias)
        i = jax.nn.sigmoid(gates[:, 0:C])
        f = jax.nn.sigmoid(gates[:, C:2 * C])
        g = jnp.tanh(gates[:, 2 * C:3 * C])
        o = jax.nn.sigmoid(gates[:, 3 * C:4 * C])
        cst = f * cst + i * g
        hst = o * jnp.tanh(cst)
        hseq_ref[t] = hst
        p_ref[0, t] = jnp.sum(jnp.exp(hst), axis=0)


def _k3(acc2s, den2s, b2, WihT, WhhT, bih, bhh):
    in_specs = (
        [pl.BlockSpec((NB, D), lambda i: (i, 0)) for _ in range(T)]
        + [pl.BlockSpec((2, NB, 16), lambda i: (0, i, 0)) for _ in range(T)]
        + [
            pl.BlockSpec((D,), lambda i: (0,)),
            pl.BlockSpec((D, 4 * C), lambda i: (0, 0)),
            pl.BlockSpec((C, 4 * C), lambda i: (0, 0)),
            pl.BlockSpec((4 * C,), lambda i: (0,)),
            pl.BlockSpec((4 * C,), lambda i: (0,)),
        ])
    return pl.pallas_call(
        _k3_body,
        grid=(NBLK,),
        in_specs=in_specs,
        out_specs=[
            pl.BlockSpec((T, NB, C), lambda i: (0, i, 0)),
            pl.BlockSpec((1, T, C), lambda i: (i, 0, 0)),
        ],
        out_shape=[
            jax.ShapeDtypeStruct((T, N, C), jnp.float32),
            jax.ShapeDtypeStruct((NBLK, T, C), jnp.float32),
        ],
    )(*acc2s, *den2s, b2, WihT, WhhT, bih, bhh)


def _k4_body(h_ref, p_ref, o_ref):
    tot = jnp.sum(p_ref[...], axis=0)[None, :, :]
    o_ref[...] = jnp.exp(h_ref[...]) / tot.transpose(1, 0, 2)


def _k4(hseq, p):
    return pl.pallas_call(
        _k4_body,
        grid=(NBLK,),
        in_specs=[
            pl.BlockSpec((T, NB, C), lambda i: (0, i, 0)),
            pl.BlockSpec((NBLK, T, C), lambda i: (0, 0, 0)),
        ],
        out_specs=pl.BlockSpec((T, NB, C), lambda i: (0, i, 0)),
        out_shape=jax.ShapeDtypeStruct((T, N, C), jnp.float32),
    )(hseq, p)


RPT_A = NP1 // NS


def _eka_body(src_hbm, dst_hbm, as_hbm, ad_hbm, z16_hbm,
              exc_hbm, den_hbm,
              srcv, dstv, arows, brows, exbuf, den_sh, sem):
    c = lax.axis_index("c")
    s = lax.axis_index("s")
    wid = s * NC + c
    pltpu.sync_copy(z16_hbm, den_sh.at[pl.ds(s * RPT_A, RPT_A)])
    plsc.subcore_barrier()

    def chunk(k, carry):
        ch = wid * CPT_A + k
        base = ch * KCH
        pltpu.sync_copy(src_hbm.at[pl.ds(base, KCH)], srcv)
        pltpu.sync_copy(dst_hbm.at[pl.ds(base, KCH)], dstv)
        pltpu.async_copy(as_hbm.at[srcv], arows, sem).wait()
        pltpu.async_copy(ad_hbm.at[dstv], brows, sem).wait()

        def row(i, carry2):
            e = arows[i] + brows[i]
            e = jnp.where(e >= 0.0, e, 0.2 * e)
            exbuf[i] = jnp.exp(e)
            return carry2

        lax.fori_loop(0, KCH, row, 0)
        pltpu.sync_copy(exbuf, den_sh.at[dstv], add=True)
        pltpu.sync_copy(exbuf, exc_hbm.at[pl.ds(base, KCH)])
        return carry

    lax.fori_loop(0, CPT_A, chunk, 0)
    plsc.subcore_barrier()
    pltpu.sync_copy(den_sh.at[pl.ds(s * RPT_A, RPT_A)],
                    den_hbm.at[c].at[pl.ds(s * RPT_A, RPT_A)])


_SC_PARAMS = pltpu.CompilerParams(use_tc_tiling_on_sc=False)

_EKA = pl.kernel(
    _eka_body,
    out_type=[jax.ShapeDtypeStruct((ETP, 16), jnp.float32),
              jax.ShapeDtypeStruct((2, NP1, 16), jnp.float32)],
    mesh=_sc_mesh,
    compiler_params=_SC_PARAMS,
    scratch_types=[
        pltpu.VMEM((KCH,), jnp.int32),
        pltpu.VMEM((KCH,), jnp.int32),
        pltpu.VMEM((KCH, 16), jnp.float32),
        pltpu.VMEM((KCH, 16), jnp.float32),
        pltpu.VMEM((KCH, 16), jnp.float32),
        pltpu.VMEM_SHARED((NP1, 16), jnp.float32),
        pltpu.SemaphoreType.DMA,
    ],
)


def _ekb_body(nh, src_hbm, dst_hbm, hflat_hbm, exc_hbm, zrow_hbm, acc_hbm,
              srcv, dstv, idxv, exv2, hbuf, acc_sh, sem):
    c = lax.axis_index("c")
    s = lax.axis_index("s")
    rounds = H if nh == 8 else 1
    npart = NP1 if nh == 8 else NH2
    rpt = npart // NS

    for h in range(rounds):
        pltpu.sync_copy(zrow_hbm.at[pl.ds(0, rpt)],
                        acc_sh.at[pl.ds(s * rpt, rpt)])
        plsc.subcore_barrier()

        def chunk(k, carry):
            if nh == 8:
                ch = (s * NC + c) * CPT_A + k
            else:
                ch = s * CPT_B + k
            base = ch * KCH
            pltpu.sync_copy(src_hbm.at[pl.ds(base, KCH)], srcv)
            pltpu.sync_copy(dst_hbm.at[pl.ds(base, KCH)], dstv)
            pltpu.sync_copy(exc_hbm.at[pl.ds(base, KCH)], exv2)
            if nh == 8:
                off = h * N
                for v in range(KCH // L):
                    idxv[pl.ds(v * L, L)] = srcv[pl.ds(v * L, L)] + off
                gsrc, dref = idxv, dstv
            else:
                noff = c * 5000
                for v in range(KCH // L):
                    dl = dstv[pl.ds(v * L, L)] - noff
                    ok = (dl >= 0) & (dl < 5000)
                    idxv[pl.ds(v * L, L)] = jnp.where(ok, dl, 5000)
                gsrc, dref = srcv, idxv
            pltpu.async_copy(hflat_hbm.at[gsrc], hbuf, sem).wait()

            def srow(i, carry2):
                sc = exv2[i][h]
                for v in range(D // L):
                    hbuf[i, pl.ds(v * L, L)] = hbuf[i, pl.ds(v * L, L)] * sc
                return carry2

            lax.fori_loop(0, KCH, srow, 0)
            pltpu.sync_copy(hbuf, acc_sh.at[dref], add=True)
            return carry

        lax.fori_loop(0, CPT_A if nh == 8 else CPT_B, chunk, 0)
        plsc.subcore_barrier()
        if nh == 8:
            nwb = N // NS
            pltpu.sync_copy(acc_sh.at[pl.ds(s * nwb, nwb)],
                            acc_hbm.at[c].at[pl.ds(h * N + s * nwb, nwb)])
        else:
            nwb = NH2 // NS
            pltpu.sync_copy(acc_sh.at[pl.ds(s * nwb, nwb)],
                            acc_hbm.at[pl.ds(c * NH2 + s * nwb, nwb)])
        if h + 1 < rounds:
            plsc.subcore_barrier()


def _make_ekb(nh):
    if nh == 8:
        out_type = jax.ShapeDtypeStruct((2, H * N, D), jnp.float32)
    else:
        out_type = jax.ShapeDtypeStruct((NC * NH2, D), jnp.float32)
    return pl.kernel(
        functools.partial(_ekb_body, nh),
        out_type=out_type,
        mesh=_sc_mesh,
        compiler_params=_SC_PARAMS,
        scratch_types=[
            pltpu.VMEM((KCH,), jnp.int32),
            pltpu.VMEM((KCH,), jnp.int32),
            pltpu.VMEM((KCH,), jnp.int32),
            pltpu.VMEM((KCH, 16), jnp.float32),
            pltpu.VMEM((KCH, D), jnp.float32),
            pltpu.VMEM_SHARED((NP1, D), jnp.float32),
            pltpu.SemaphoreType.DMA,
        ],
    )


_EKB8 = _make_ekb(8)
_EKB1 = _make_ekb(1)


def _edge_layer(h_flat, s_tab, d_tab, srcp, dstp, z16, zrow, nh):
    zpad = jnp.zeros((NP1 - N, 16), jnp.float32)
    s_pad = jnp.concatenate([s_tab, zpad], axis=0)
    d_pad = jnp.concatenate([d_tab, zpad], axis=0)
    exc, den = _EKA(srcp, dstp, s_pad, d_pad, z16)
    if nh == 8:
        acc = _EKB8(srcp, dstp, h_flat, exc, zrow)
    else:
        accp = _EKB1(srcp, dstp, h_flat, exc, zrow)
        acc = jnp.concatenate([accp[:5000], accp[NH2:NH2 + 5000]], axis=0)
    return acc, den


def kernel(x, edge_attr, W1, a_src1, a_dst1, b1, W2, a_src2, a_dst2, b2,
           Wih, Whh, bih, bhh, edge_index):
    loop = jnp.arange(N, dtype=edge_index.dtype)
    pad = ETP - (E + N)
    srcp = jnp.concatenate([edge_index[0], loop,
                            jnp.zeros((pad,), edge_index.dtype)])
    dstp = jnp.concatenate([edge_index[1], loop,
                            jnp.full((pad,), N, edge_index.dtype)])
    z16 = jnp.zeros((RPT_A, 16), jnp.float32)
    zrow = jnp.zeros((NP1 // NS, D), jnp.float32)

    acc2s, den2s = [], []
    for t in range(T):
        h1, s1, d1 = _k1(x[t], W1, a_src1, a_dst1)
        acc1, den1 = _edge_layer(h1.reshape(H * N, D), s1, d1,
                                 srcp, dstp, z16, zrow, H)
        h2, s2, d2 = _k2(acc1.reshape(2, H, N, D), den1, b1, W2,
                         a_src2, a_dst2)
        acc2, den2 = _edge_layer(h2, s2, d2, srcp, dstp, z16, zrow, 1)
        acc2s.append(acc2)
        den2s.append(den2)

    hseq, p = _k3(acc2s, den2s, b2, Wih.T, Whh.T, bih, bhh)
    return _k4(hseq, p)


# speedup vs baseline: 12.4462x; 1.2279x over previous
import functools

import jax
import jax.numpy as jnp
from jax import lax
from jax.experimental import pallas as pl
from jax.experimental.pallas import tpu as pltpu
from jax.experimental.pallas import tpu_sc as plsc

N = 10000
E = 320000
D = 128
H = 8
T = 4
C = 16
NB = 400
NBLK = N // NB

NC, NS, L = 2, 16, 16
KCH = 128
ETP = 331776
NCH = ETP // KCH
CPT_A = NCH // (NC * NS)
CPT_B = NCH // NS
NP1 = 10240
NH2 = 5120

_sc_mesh = plsc.VectorSubcoreMesh(core_axis_name="c", subcore_axis_name="s")


def _k1_body(x_ref, w1_ref, as_ref, ad_ref, h1_ref, s_ref, d_ref):
    xb = x_ref[...]
    s_ref[...] = jnp.zeros((NB, 16), jnp.float32)
    d_ref[...] = jnp.zeros((NB, 16), jnp.float32)
    for h in range(H):
        w = w1_ref[:, h * D:(h + 1) * D]
        hh = jnp.dot(xb, w, preferred_element_type=jnp.float32)
        h1_ref[h] = hh
        s_ref[:, h:h + 1] = jnp.sum(hh * as_ref[h][None, :], axis=1)[:, None]
        d_ref[:, h:h + 1] = jnp.sum(hh * ad_ref[h][None, :], axis=1)[:, None]


def _k1(x_t, W1, a_src1, a_dst1):
    return pl.pallas_call(
        _k1_body,
        grid=(NBLK,),
        in_specs=[
            pl.BlockSpec((NB, D), lambda i: (i, 0)),
            pl.BlockSpec((D, H * D), lambda i: (0, 0)),
            pl.BlockSpec((H, D), lambda i: (0, 0)),
            pl.BlockSpec((H, D), lambda i: (0, 0)),
        ],
        out_specs=[
            pl.BlockSpec((H, NB, D), lambda i: (0, i, 0)),
            pl.BlockSpec((NB, 16), lambda i: (i, 0)),
            pl.BlockSpec((NB, 16), lambda i: (i, 0)),
        ],
        out_shape=[
            jax.ShapeDtypeStruct((H, N, D), jnp.float32),
            jax.ShapeDtypeStruct((N, 16), jnp.float32),
            jax.ShapeDtypeStruct((N, 16), jnp.float32),
        ],
    )(x_t, W1, a_src1, a_dst1)


def _k2_body(acc_ref, den_ref, b1_ref, w2_ref, as2_ref, ad2_ref,
             h2_ref, s_ref, d_ref):
    den = den_ref[0] + den_ref[1]
    acc = jnp.zeros((NB, D), dtype=jnp.float32)
    for h in range(H):
        z = ((acc_ref[0, h] + acc_ref[1, h]) / den[:, h:h + 1]
             + b1_ref[pl.ds(h * D, D)][None, :])
        z = jnp.maximum(z, 0.0)
        acc = acc + jnp.dot(z, w2_ref[pl.ds(h * D, D), :],
                            preferred_element_type=jnp.float32)
    h2_ref[...] = acc
    s_ref[...] = jnp.zeros((NB, 16), jnp.float32)
    d_ref[...] = jnp.zeros((NB, 16), jnp.float32)
    s_ref[:, 0:1] = jnp.sum(acc * as2_ref[0][None, :], axis=1)[:, None]
    d_ref[:, 0:1] = jnp.sum(acc * ad2_ref[0][None, :], axis=1)[:, None]


def _k2(acc1, den1, b1, W2, a_src2, a_dst2):
    return pl.pallas_call(
        _k2_body,
        grid=(NBLK,),
        in_specs=[
            pl.BlockSpec((2, H, NB, D), lambda i: (0, 0, i, 0)),
            pl.BlockSpec((2, NB, 16), lambda i: (0, i, 0)),
            pl.BlockSpec((H * D,), lambda i: (0,)),
            pl.BlockSpec((H * D, D), lambda i: (0, 0)),
            pl.BlockSpec((1, D), lambda i: (0, 0)),
            pl.BlockSpec((1, D), lambda i: (0, 0)),
        ],
        out_specs=[
            pl.BlockSpec((NB, D), lambda i: (i, 0)),
            pl.BlockSpec((NB, 16), lambda i: (i, 0)),
            pl.BlockSpec((NB, 16), lambda i: (i, 0)),
        ],
        out_shape=[
            jax.ShapeDtypeStruct((N, D), jnp.float32),
            jax.ShapeDtypeStruct((N, 16), jnp.float32),
            jax.ShapeDtypeStruct((N, 16), jnp.float32),
        ],
    )(acc1, den1, b1, W2, a_src2, a_dst2)


def _k3_body(*refs):
    acc_refs = refs[0:T]
    den_refs = refs[T:2 * T]
    b2_ref, wih_ref, whh_ref, bi_ref, bh_ref = refs[2 * T:2 * T + 5]
    hseq_ref, p_ref = refs[2 * T + 5:]
    hst = jnp.zeros((NB, C), dtype=jnp.float32)
    cst = jnp.zeros((NB, C), dtype=jnp.float32)
    bias = (bi_ref[...] + bh_ref[...])[None, :]
    for t in range(T):
        den = den_refs[t][0] + den_refs[t][1]
        hs = acc_refs[t][...] / den[:, 0:1] + b2_ref[...][None, :]
        gates = (jnp.dot(hs, wih_ref[...], preferred_element_type=jnp.float32)
                 + jnp.dot(hst, whh_ref[...], preferred_element_type=jnp.float32)
                 + bias)
        i = jax.nn.sigmoid(gates[:, 0:C])
        f = jax.nn.sigmoid(gates[:, C:2 * C])
        g = jnp.tanh(gates[:, 2 * C:3 * C])
        o = jax.nn.sigmoid(gates[:, 3 * C:4 * C])
        cst = f * cst + i * g
        hst = o * jnp.tanh(cst)
        hseq_ref[t] = hst
        p_ref[0, t] = jnp.sum(jnp.exp(hst), axis=0)


def _k3(acc2s, den2s, b2, WihT, WhhT, bih, bhh):
    in_specs = (
        [pl.BlockSpec((NB, D), lambda i: (i, 0)) for _ in range(T)]
        + [pl.BlockSpec((2, NB, 16), lambda i: (0, i, 0)) for _ in range(T)]
        + [
            pl.BlockSpec((D,), lambda i: (0,)),
            pl.BlockSpec((D, 4 * C), lambda i: (0, 0)),
            pl.BlockSpec((C, 4 * C), lambda i: (0, 0)),
            pl.BlockSpec((4 * C,), lambda i: (0,)),
            pl.BlockSpec((4 * C,), lambda i: (0,)),
        ])
    return pl.pallas_call(
        _k3_body,
        grid=(NBLK,),
        in_specs=in_specs,
        out_specs=[
            pl.BlockSpec((T, NB, C), lambda i: (0, i, 0)),
            pl.BlockSpec((1, T, C), lambda i: (i, 0, 0)),
        ],
        out_shape=[
            jax.ShapeDtypeStruct((T, N, C), jnp.float32),
            jax.ShapeDtypeStruct((NBLK, T, C), jnp.float32),
        ],
    )(*acc2s, *den2s, b2, WihT, WhhT, bih, bhh)


def _k4_body(h_ref, p_ref, o_ref):
    tot = jnp.sum(p_ref[...], axis=0)[None, :, :]
    o_ref[...] = jnp.exp(h_ref[...]) / tot.transpose(1, 0, 2)


def _k4(hseq, p):
    return pl.pallas_call(
        _k4_body,
        grid=(NBLK,),
        in_specs=[
            pl.BlockSpec((T, NB, C), lambda i: (0, i, 0)),
            pl.BlockSpec((NBLK, T, C), lambda i: (0, 0, 0)),
        ],
        out_specs=pl.BlockSpec((T, NB, C), lambda i: (0, i, 0)),
        out_shape=jax.ShapeDtypeStruct((T, N, C), jnp.float32),
    )(hseq, p)


RPT_A = NP1 // NS


def _eka_body(src_hbm, dst_hbm, as_hbm, ad_hbm, z16_hbm,
              exc_hbm, den_hbm,
              srcv, dstv, arows, brows, exbuf, den_sh, sem):
    c = lax.axis_index("c")
    s = lax.axis_index("s")
    wid = s * NC + c
    pltpu.sync_copy(z16_hbm, den_sh.at[pl.ds(s * RPT_A, RPT_A)])
    plsc.subcore_barrier()

    def chunk(k, carry):
        ch = wid * CPT_A + k
        base = ch * KCH
        pltpu.sync_copy(src_hbm.at[pl.ds(base, KCH)], srcv)
        pltpu.sync_copy(dst_hbm.at[pl.ds(base, KCH)], dstv)
        pltpu.async_copy(as_hbm.at[srcv], arows, sem).wait()
        pltpu.async_copy(ad_hbm.at[dstv], brows, sem).wait()

        def row(i, carry2):
            e = arows[i] + brows[i]
            e = jnp.where(e >= 0.0, e, 0.2 * e)
            exbuf[i] = jnp.exp(e)
            return carry2

        lax.fori_loop(0, KCH, row, 0)
        pltpu.sync_copy(exbuf, den_sh.at[dstv], add=True)
        pltpu.sync_copy(exbuf, exc_hbm.at[pl.ds(base, KCH)])
        return carry

    lax.fori_loop(0, CPT_A, chunk, 0)
    plsc.subcore_barrier()
    pltpu.sync_copy(den_sh.at[pl.ds(s * RPT_A, RPT_A)],
                    den_hbm.at[c].at[pl.ds(s * RPT_A, RPT_A)])


_SC_PARAMS = pltpu.CompilerParams(use_tc_tiling_on_sc=False)

_EKA = pl.kernel(
    _eka_body,
    out_type=[jax.ShapeDtypeStruct((ETP, 16), jnp.float32),
              jax.ShapeDtypeStruct((2, NP1, 16), jnp.float32)],
    mesh=_sc_mesh,
    compiler_params=_SC_PARAMS,
    scratch_types=[
        pltpu.VMEM((KCH,), jnp.int32),
        pltpu.VMEM((KCH,), jnp.int32),
        pltpu.VMEM((KCH, 16), jnp.float32),
        pltpu.VMEM((KCH, 16), jnp.float32),
        pltpu.VMEM((KCH, 16), jnp.float32),
        pltpu.VMEM_SHARED((NP1, 16), jnp.float32),
        pltpu.SemaphoreType.DMA,
    ],
)


def _ekb_body(nh, src_hbm, dst_hbm, hflat_hbm, exc_hbm, zrow_hbm, acc_hbm,
              srcv, dstv, idxv, exv2, hbuf,
              srcw, dstw, idxw, exw2, hbuf2, acc_sh, sem, sem2):
    c = lax.axis_index("c")
    s = lax.axis_index("s")
    rounds = H if nh == 8 else 1
    npart = NP1 if nh == 8 else NH2
    rpt = npart // NS
    cpt = CPT_A if nh == 8 else CPT_B

    for h in range(rounds):
        pltpu.sync_copy(zrow_hbm.at[pl.ds(0, rpt)],
                        acc_sh.at[pl.ds(s * rpt, rpt)])
        plsc.subcore_barrier()

        def issue(ch, sv, dv, iv, ev, hb, sm):
            base = ch * KCH
            pltpu.sync_copy(src_hbm.at[pl.ds(base, KCH)], sv)
            pltpu.sync_copy(dst_hbm.at[pl.ds(base, KCH)], dv)
            pltpu.sync_copy(exc_hbm.at[pl.ds(base, KCH)], ev)
            if nh == 8:
                off = h * N
                for v in range(KCH // L):
                    iv[pl.ds(v * L, L)] = sv[pl.ds(v * L, L)] + off
                gsrc = iv
            else:
                noff = c * 5000
                for v in range(KCH // L):
                    dl = dv[pl.ds(v * L, L)] - noff
                    ok = (dl >= 0) & (dl < 5000)
                    iv[pl.ds(v * L, L)] = jnp.where(ok, dl, 5000)
                gsrc = sv
            return pltpu.async_copy(hflat_hbm.at[gsrc], hb, sm)

        def finish(cp, dv, iv, ev, hb):
            cp.wait()

            def srow(i, carry2):
                sc = ev[i][h]
                for v in range(D // L):
                    hb[i, pl.ds(v * L, L)] = hb[i, pl.ds(v * L, L)] * sc
                return carry2

            lax.fori_loop(0, KCH, srow, 0)
            dref = dv if nh == 8 else iv
            pltpu.sync_copy(hb, acc_sh.at[dref], add=True)

        def chunk_id(k):
            if nh == 8:
                return (s * NC + c) * CPT_A + k
            return s * CPT_B + k

        def pair(j, carry):
            k = 2 * j
            cpa = issue(chunk_id(k), srcv, dstv, idxv, exv2, hbuf, sem)
            cpb = issue(chunk_id(k + 1), srcw, dstw, idxw, exw2, hbuf2, sem2)
            finish(cpa, dstv, idxv, exv2, hbuf)
            finish(cpb, dstw, idxw, exw2, hbuf2)
            return carry

        lax.fori_loop(0, cpt // 2, pair, 0)
        if cpt % 2:
            cpa = issue(chunk_id(cpt - 1), srcv, dstv, idxv, exv2, hbuf, sem)
            finish(cpa, dstv, idxv, exv2, hbuf)
        plsc.subcore_barrier()
        if nh == 8:
            nwb = N // NS
            pltpu.sync_copy(acc_sh.at[pl.ds(s * nwb, nwb)],
                            acc_hbm.at[c].at[pl.ds(h * N + s * nwb, nwb)])
        else:
            nwb = NH2 // NS
            pltpu.sync_copy(acc_sh.at[pl.ds(s * nwb, nwb)],
                            acc_hbm.at[pl.ds(c * NH2 + s * nwb, nwb)])
        if h + 1 < rounds:
            plsc.subcore_barrier()


def _make_ekb(nh):
    if nh == 8:
        out_type = jax.ShapeDtypeStruct((2, H * N, D), jnp.float32)
    else:
        out_type = jax.ShapeDtypeStruct((NC * NH2, D), jnp.float32)
    return pl.kernel(
        functools.partial(_ekb_body, nh),
        out_type=out_type,
        mesh=_sc_mesh,
        compiler_params=_SC_PARAMS,
        scratch_types=[
            pltpu.VMEM((KCH,), jnp.int32),
            pltpu.VMEM((KCH,), jnp.int32),
            pltpu.VMEM((KCH,), jnp.int32),
            pltpu.VMEM((KCH, 16), jnp.float32),
            pltpu.VMEM((KCH, D), jnp.float32),
            pltpu.VMEM((KCH,), jnp.int32),
            pltpu.VMEM((KCH,), jnp.int32),
            pltpu.VMEM((KCH,), jnp.int32),
            pltpu.VMEM((KCH, 16), jnp.float32),
            pltpu.VMEM((KCH, D), jnp.float32),
            pltpu.VMEM_SHARED((NP1, D), jnp.float32),
            pltpu.SemaphoreType.DMA,
            pltpu.SemaphoreType.DMA,
        ],
    )


_EKB8 = _make_ekb(8)
_EKB1 = _make_ekb(1)


def _edge_layer(h_flat, s_tab, d_tab, srcp, dstp, z16, zrow, nh):
    zpad = jnp.zeros((NP1 - N, 16), jnp.float32)
    s_pad = jnp.concatenate([s_tab, zpad], axis=0)
    d_pad = jnp.concatenate([d_tab, zpad], axis=0)
    exc, den = _EKA(srcp, dstp, s_pad, d_pad, z16)
    if nh == 8:
        acc = _EKB8(srcp, dstp, h_flat, exc, zrow)
    else:
        accp = _EKB1(srcp, dstp, h_flat, exc, zrow)
        acc = jnp.concatenate([accp[:5000], accp[NH2:NH2 + 5000]], axis=0)
    return acc, den


def kernel(x, edge_attr, W1, a_src1, a_dst1, b1, W2, a_src2, a_dst2, b2,
           Wih, Whh, bih, bhh, edge_index):
    loop = jnp.arange(N, dtype=edge_index.dtype)
    pad = ETP - (E + N)
    srcp = jnp.concatenate([edge_index[0], loop,
                            jnp.zeros((pad,), edge_index.dtype)])
    dstp = jnp.concatenate([edge_index[1], loop,
                            jnp.full((pad,), N, edge_index.dtype)])
    z16 = jnp.zeros((RPT_A, 16), jnp.float32)
    zrow = jnp.zeros((NP1 // NS, D), jnp.float32)

    acc2s, den2s = [], []
    for t in range(T):
        h1, s1, d1 = _k1(x[t], W1, a_src1, a_dst1)
        acc1, den1 = _edge_layer(h1.reshape(H * N, D), s1, d1,
                                 srcp, dstp, z16, zrow, H)
        h2, s2, d2 = _k2(acc1.reshape(2, H, N, D), den1, b1, W2,
                         a_src2, a_dst2)
        acc2, den2 = _edge_layer(h2, s2, d2, srcp, dstp, z16, zrow, 1)
        acc2s.append(acc2)
        den2s.append(den2)

    hseq, p = _k3(acc2s, den2s, b2, Wih.T, Whh.T, bih, bhh)
    return _k4(hseq, p)


# speedup vs baseline: 13.1331x; 1.0552x over previous
import functools

import jax
import jax.numpy as jnp
from jax import lax
from jax.experimental import pallas as pl
from jax.experimental.pallas import tpu as pltpu
from jax.experimental.pallas import tpu_sc as plsc

N = 10000
E = 320000
D = 128
H = 8
T = 4
C = 16
NB = 400
NBLK = N // NB

NC, NS, L = 2, 16, 16
KCH = 128
ETP = 331776
NCH = ETP // KCH
CPT_A = NCH // (NC * NS)
CPT_B = NCH // NS
NP1 = 10240
NH2 = 5120

_sc_mesh = plsc.VectorSubcoreMesh(core_axis_name="c", subcore_axis_name="s")


def _k1_body(x_ref, w1_ref, as_ref, ad_ref, h1_ref, s_ref, d_ref):
    xb = x_ref[...]
    s_ref[...] = jnp.zeros((NB, 16), jnp.float32)
    d_ref[...] = jnp.zeros((NB, 16), jnp.float32)
    for h in range(H):
        w = w1_ref[:, h * D:(h + 1) * D]
        hh = jnp.dot(xb, w, preferred_element_type=jnp.float32)
        h1_ref[h] = hh
        s_ref[:, h:h + 1] = jnp.sum(hh * as_ref[h][None, :], axis=1)[:, None]
        d_ref[:, h:h + 1] = jnp.sum(hh * ad_ref[h][None, :], axis=1)[:, None]


def _k1(x_t, W1, a_src1, a_dst1):
    return pl.pallas_call(
        _k1_body,
        grid=(NBLK,),
        in_specs=[
            pl.BlockSpec((NB, D), lambda i: (i, 0)),
            pl.BlockSpec((D, H * D), lambda i: (0, 0)),
            pl.BlockSpec((H, D), lambda i: (0, 0)),
            pl.BlockSpec((H, D), lambda i: (0, 0)),
        ],
        out_specs=[
            pl.BlockSpec((H, NB, D), lambda i: (0, i, 0)),
            pl.BlockSpec((NB, 16), lambda i: (i, 0)),
            pl.BlockSpec((NB, 16), lambda i: (i, 0)),
        ],
        out_shape=[
            jax.ShapeDtypeStruct((H, N, D), jnp.float32),
            jax.ShapeDtypeStruct((N, 16), jnp.float32),
            jax.ShapeDtypeStruct((N, 16), jnp.float32),
        ],
    )(x_t, W1, a_src1, a_dst1)


def _k2_body(acc_ref, den_ref, b1_ref, w2_ref, as2_ref, ad2_ref,
             h2_ref, s_ref, d_ref):
    den = den_ref[0] + den_ref[1]
    acc = jnp.zeros((NB, D), dtype=jnp.float32)
    for h in range(H):
        z = ((acc_ref[0, h] + acc_ref[1, h]) / den[:, h:h + 1]
             + b1_ref[pl.ds(h * D, D)][None, :])
        z = jnp.maximum(z, 0.0)
        acc = acc + jnp.dot(z, w2_ref[pl.ds(h * D, D), :],
                            preferred_element_type=jnp.float32)
    h2_ref[...] = acc
    s_ref[...] = jnp.zeros((NB, 16), jnp.float32)
    d_ref[...] = jnp.zeros((NB, 16), jnp.float32)
    s_ref[:, 0:1] = jnp.sum(acc * as2_ref[0][None, :], axis=1)[:, None]
    d_ref[:, 0:1] = jnp.sum(acc * ad2_ref[0][None, :], axis=1)[:, None]


def _k2(acc1, den1, b1, W2, a_src2, a_dst2):
    return pl.pallas_call(
        _k2_body,
        grid=(NBLK,),
        in_specs=[
            pl.BlockSpec((2, H, NB, D), lambda i: (0, 0, i, 0)),
            pl.BlockSpec((2, NB, 16), lambda i: (0, i, 0)),
            pl.BlockSpec((H * D,), lambda i: (0,)),
            pl.BlockSpec((H * D, D), lambda i: (0, 0)),
            pl.BlockSpec((1, D), lambda i: (0, 0)),
            pl.BlockSpec((1, D), lambda i: (0, 0)),
        ],
        out_specs=[
            pl.BlockSpec((NB, D), lambda i: (i, 0)),
            pl.BlockSpec((NB, 16), lambda i: (i, 0)),
            pl.BlockSpec((NB, 16), lambda i: (i, 0)),
        ],
        out_shape=[
            jax.ShapeDtypeStruct((N, D), jnp.float32),
            jax.ShapeDtypeStruct((N, 16), jnp.float32),
            jax.ShapeDtypeStruct((N, 16), jnp.float32),
        ],
    )(acc1, den1, b1, W2, a_src2, a_dst2)


def _k3_body(*refs):
    acc_refs = refs[0:T]
    den_refs = refs[T:2 * T]
    b2_ref, wih_ref, whh_ref, bi_ref, bh_ref = refs[2 * T:2 * T + 5]
    hseq_ref, p_ref = refs[2 * T + 5:]
    hst = jnp.zeros((NB, C), dtype=jnp.float32)
    cst = jnp.zeros((NB, C), dtype=jnp.float32)
    bias = (bi_ref[...] + bh_ref[...])[None, :]
    for t in range(T):
        den = den_refs[t][0] + den_refs[t][1]
        hs = acc_refs[t][...] / den[:, 0:1] + b2_ref[...][None, :]
        gates = (jnp.dot(hs, wih_ref[...], preferred_element_type=jnp.float32)
                 + jnp.dot(hst, whh_ref[...], preferred_element_type=jnp.float32)
                 + bias)
        i = jax.nn.sigmoid(gates[:, 0:C])
        f = jax.nn.sigmoid(gates[:, C:2 * C])
        g = jnp.tanh(gates[:, 2 * C:3 * C])
        o = jax.nn.sigmoid(gates[:, 3 * C:4 * C])
        cst = f * cst + i * g
        hst = o * jnp.tanh(cst)
        hseq_ref[t] = hst
        p_ref[0, t] = jnp.sum(jnp.exp(hst), axis=0)


def _k3(acc2s, den2s, b2, WihT, WhhT, bih, bhh):
    in_specs = (
        [pl.BlockSpec((NB, D), lambda i: (i, 0)) for _ in range(T)]
        + [pl.BlockSpec((2, NB, 16), lambda i: (0, i, 0)) for _ in range(T)]
        + [
            pl.BlockSpec((D,), lambda i: (0,)),
            pl.BlockSpec((D, 4 * C), lambda i: (0, 0)),
            pl.BlockSpec((C, 4 * C), lambda i: (0, 0)),
            pl.BlockSpec((4 * C,), lambda i: (0,)),
            pl.BlockSpec((4 * C,), lambda i: (0,)),
        ])
    return pl.pallas_call(
        _k3_body,
        grid=(NBLK,),
        in_specs=in_specs,
        out_specs=[
            pl.BlockSpec((T, NB, C), lambda i: (0, i, 0)),
            pl.BlockSpec((1, T, C), lambda i: (i, 0, 0)),
        ],
        out_shape=[
            jax.ShapeDtypeStruct((T, N, C), jnp.float32),
            jax.ShapeDtypeStruct((NBLK, T, C), jnp.float32),
        ],
    )(*acc2s, *den2s, b2, WihT, WhhT, bih, bhh)


def _k4_body(h_ref, p_ref, o_ref):
    tot = jnp.sum(p_ref[...], axis=0)[None, :, :]
    o_ref[...] = jnp.exp(h_ref[...]) / tot.transpose(1, 0, 2)


def _k4(hseq, p):
    return pl.pallas_call(
        _k4_body,
        grid=(NBLK,),
        in_specs=[
            pl.BlockSpec((T, NB, C), lambda i: (0, i, 0)),
            pl.BlockSpec((NBLK, T, C), lambda i: (0, 0, 0)),
        ],
        out_specs=pl.BlockSpec((T, NB, C), lambda i: (0, i, 0)),
        out_shape=jax.ShapeDtypeStruct((T, N, C), jnp.float32),
    )(hseq, p)


RPT_A = NP1 // NS


def _eka_body(src_hbm, dst_hbm, as_hbm, ad_hbm, z16_hbm,
              exc_hbm, den_hbm,
              srcv, dstv, arows, brows, exbuf,
              srcw, dstw, arows2, brows2, exbuf2, den_sh, sem, sem2):
    c = lax.axis_index("c")
    s = lax.axis_index("s")
    wid = s * NC + c
    pltpu.sync_copy(z16_hbm, den_sh.at[pl.ds(s * RPT_A, RPT_A)])
    plsc.subcore_barrier()

    def issue(k, sv, dv, ar, br, sm):
        base = (wid * CPT_A + k) * KCH
        pltpu.sync_copy(src_hbm.at[pl.ds(base, KCH)], sv)
        pltpu.sync_copy(dst_hbm.at[pl.ds(base, KCH)], dv)
        cpa = pltpu.async_copy(as_hbm.at[sv], ar, sm)
        cpb = pltpu.async_copy(ad_hbm.at[dv], br, sm)
        return cpa, cpb

    def finish(k, cps, dv, ar, br, eb):
        cps[0].wait()
        cps[1].wait()

        def row(i, carry2):
            e = ar[i] + br[i]
            e = jnp.where(e >= 0.0, e, 0.2 * e)
            eb[i] = jnp.exp(e)
            return carry2

        lax.fori_loop(0, KCH, row, 0)
        base = (wid * CPT_A + k) * KCH
        pltpu.sync_copy(eb, den_sh.at[dv], add=True)
        pltpu.sync_copy(eb, exc_hbm.at[pl.ds(base, KCH)])

    def pair(j, carry):
        k = 2 * j
        cpsa = issue(k, srcv, dstv, arows, brows, sem)
        cpsb = issue(k + 1, srcw, dstw, arows2, brows2, sem2)
        finish(k, cpsa, dstv, arows, brows, exbuf)
        finish(k + 1, cpsb, dstw, arows2, brows2, exbuf2)
        return carry

    lax.fori_loop(0, CPT_A // 2, pair, 0)
    if CPT_A % 2:
        cpsa = issue(CPT_A - 1, srcv, dstv, arows, brows, sem)
        finish(CPT_A - 1, cpsa, dstv, arows, brows, exbuf)
    plsc.subcore_barrier()
    pltpu.sync_copy(den_sh.at[pl.ds(s * RPT_A, RPT_A)],
                    den_hbm.at[c].at[pl.ds(s * RPT_A, RPT_A)])


_SC_PARAMS = pltpu.CompilerParams(use_tc_tiling_on_sc=False)

_EKA = pl.kernel(
    _eka_body,
    out_type=[jax.ShapeDtypeStruct((ETP, 16), jnp.float32),
              jax.ShapeDtypeStruct((2, NP1, 16), jnp.float32)],
    mesh=_sc_mesh,
    compiler_params=_SC_PARAMS,
    scratch_types=[
        pltpu.VMEM((KCH,), jnp.int32),
        pltpu.VMEM((KCH,), jnp.int32),
        pltpu.VMEM((KCH, 16), jnp.float32),
        pltpu.VMEM((KCH, 16), jnp.float32),
        pltpu.VMEM((KCH, 16), jnp.float32),
        pltpu.VMEM((KCH,), jnp.int32),
        pltpu.VMEM((KCH,), jnp.int32),
        pltpu.VMEM((KCH, 16), jnp.float32),
        pltpu.VMEM((KCH, 16), jnp.float32),
        pltpu.VMEM((KCH, 16), jnp.float32),
        pltpu.VMEM_SHARED((NP1, 16), jnp.float32),
        pltpu.SemaphoreType.DMA,
        pltpu.SemaphoreType.DMA,
    ],
)


def _ekb_body(nh, src_hbm, dst_hbm, hflat_hbm, exc_hbm, zrow_hbm, acc_hbm,
              srcv, dstv, idxv, exv2, hbuf,
              srcw, dstw, idxw, exw2, hbuf2, acc_sh, sem, sem2):
    c = lax.axis_index("c")
    s = lax.axis_index("s")
    rounds = H if nh == 8 else 1
    npart = NP1 if nh == 8 else NH2
    rpt = npart // NS
    cpt = CPT_A if nh == 8 else CPT_B

    for h in range(rounds):
        pltpu.sync_copy(zrow_hbm.at[pl.ds(0, rpt)],
                        acc_sh.at[pl.ds(s * rpt, rpt)])
        plsc.subcore_barrier()

        def issue(ch, sv, dv, iv, ev, hb, sm):
            base = ch * KCH
            pltpu.sync_copy(src_hbm.at[pl.ds(base, KCH)], sv)
            pltpu.sync_copy(dst_hbm.at[pl.ds(base, KCH)], dv)
            pltpu.sync_copy(exc_hbm.at[pl.ds(base, KCH)], ev)
            if nh == 8:
                off = h * N
                for v in range(KCH // L):
                    iv[pl.ds(v * L, L)] = sv[pl.ds(v * L, L)] + off
                gsrc = iv
            else:
                noff = c * 5000
                for v in range(KCH // L):
                    dl = dv[pl.ds(v * L, L)] - noff
                    ok = (dl >= 0) & (dl < 5000)
                    iv[pl.ds(v * L, L)] = jnp.where(ok, dl, 5000)
                gsrc = sv
            return pltpu.async_copy(hflat_hbm.at[gsrc], hb, sm)

        def finish(cp, dv, iv, ev, hb):
            cp.wait()

            def srow(i, carry2):
                sc = ev[i][h]
                for v in range(D // L):
                    hb[i, pl.ds(v * L, L)] = hb[i, pl.ds(v * L, L)] * sc
                return carry2

            lax.fori_loop(0, KCH, srow, 0)
            dref = dv if nh == 8 else iv
            pltpu.sync_copy(hb, acc_sh.at[dref], add=True)

        def chunk_id(k):
            if nh == 8:
                return (s * NC + c) * CPT_A + k
            return s * CPT_B + k

        def pair(j, carry):
            k = 2 * j
            cpa = issue(chunk_id(k), srcv, dstv, idxv, exv2, hbuf, sem)
            cpb = issue(chunk_id(k + 1), srcw, dstw, idxw, exw2, hbuf2, sem2)
            finish(cpa, dstv, idxv, exv2, hbuf)
            finish(cpb, dstw, idxw, exw2, hbuf2)
            return carry

        lax.fori_loop(0, cpt // 2, pair, 0)
        if cpt % 2:
            cpa = issue(chunk_id(cpt - 1), srcv, dstv, idxv, exv2, hbuf, sem)
            finish(cpa, dstv, idxv, exv2, hbuf)
        plsc.subcore_barrier()
        if nh == 8:
            nwb = N // NS
            pltpu.sync_copy(acc_sh.at[pl.ds(s * nwb, nwb)],
                            acc_hbm.at[c].at[pl.ds(h * N + s * nwb, nwb)])
        else:
            nwb = NH2 // NS
            pltpu.sync_copy(acc_sh.at[pl.ds(s * nwb, nwb)],
                            acc_hbm.at[pl.ds(c * NH2 + s * nwb, nwb)])
        if h + 1 < rounds:
            plsc.subcore_barrier()


def _make_ekb(nh):
    if nh == 8:
        out_type = jax.ShapeDtypeStruct((2, H * N, D), jnp.float32)
    else:
        out_type = jax.ShapeDtypeStruct((NC * NH2, D), jnp.float32)
    return pl.kernel(
        functools.partial(_ekb_body, nh),
        out_type=out_type,
        mesh=_sc_mesh,
        compiler_params=_SC_PARAMS,
        scratch_types=[
            pltpu.VMEM((KCH,), jnp.int32),
            pltpu.VMEM((KCH,), jnp.int32),
            pltpu.VMEM((KCH,), jnp.int32),
            pltpu.VMEM((KCH, 16), jnp.float32),
            pltpu.VMEM((KCH, D), jnp.float32),
            pltpu.VMEM((KCH,), jnp.int32),
            pltpu.VMEM((KCH,), jnp.int32),
            pltpu.VMEM((KCH,), jnp.int32),
            pltpu.VMEM((KCH, 16), jnp.float32),
            pltpu.VMEM((KCH, D), jnp.float32),
            pltpu.VMEM_SHARED((NP1, D), jnp.float32),
            pltpu.SemaphoreType.DMA,
            pltpu.SemaphoreType.DMA,
        ],
    )


_EKB8 = _make_ekb(8)
_EKB1 = _make_ekb(1)


def _edge_layer(h_flat, s_tab, d_tab, srcp, dstp, z16, zrow, nh):
    zpad = jnp.zeros((NP1 - N, 16), jnp.float32)
    s_pad = jnp.concatenate([s_tab, zpad], axis=0)
    d_pad = jnp.concatenate([d_tab, zpad], axis=0)
    exc, den = _EKA(srcp, dstp, s_pad, d_pad, z16)
    if nh == 8:
        acc = _EKB8(srcp, dstp, h_flat, exc, zrow)
    else:
        accp = _EKB1(srcp, dstp, h_flat, exc, zrow)
        acc = jnp.concatenate([accp[:5000], accp[NH2:NH2 + 5000]], axis=0)
    return acc, den


def kernel(x, edge_attr, W1, a_src1, a_dst1, b1, W2, a_src2, a_dst2, b2,
           Wih, Whh, bih, bhh, edge_index):
    loop = jnp.arange(N, dtype=edge_index.dtype)
    pad = ETP - (E + N)
    srcp = jnp.concatenate([edge_index[0], loop,
                            jnp.zeros((pad,), edge_index.dtype)])
    dstp = jnp.concatenate([edge_index[1], loop,
                            jnp.full((pad,), N, edge_index.dtype)])
    z16 = jnp.zeros((RPT_A, 16), jnp.float32)
    zrow = jnp.zeros((NP1 // NS, D), jnp.float32)

    acc2s, den2s = [], []
    for t in range(T):
        h1, s1, d1 = _k1(x[t], W1, a_src1, a_dst1)
        acc1, den1 = _edge_layer(h1.reshape(H * N, D), s1, d1,
                                 srcp, dstp, z16, zrow, H)
        h2, s2, d2 = _k2(acc1.reshape(2, H, N, D), den1, b1, W2,
                         a_src2, a_dst2)
        acc2, den2 = _edge_layer(h2, s2, d2, srcp, dstp, z16, zrow, 1)
        acc2s.append(acc2)
        den2s.append(den2)

    hseq, p = _k3(acc2s, den2s, b2, Wih.T, Whh.T, bih, bhh)
    return _k4(hseq, p)
